# baseline (speedup 1.0000x reference)
import functools

import jax
import jax.numpy as jnp
from jax import lax
from jax.experimental import pallas as pl
from jax.experimental.pallas import tpu as pltpu
from jax.experimental.pallas import tpu_sc as plsc

_DIM = 64
_PAD = 128
_NBUF = 3


def _gather_body(x_hbm, emb_hbm, out_hbm, idx_v, rows_v,
                 sg0, sg1, sg2, so0, so1, so2, *, seq, n_batches, chunk0):
    wid = lax.axis_index("s") * 2 + lax.axis_index("c")
    b0 = wid * n_batches
    sem_g = (sg0, sg1, sg2)
    sem_o = (so0, so1, so2)

    pltpu.sync_copy(
        x_hbm.at[pl.ds((chunk0 + b0) * seq, n_batches * seq)], idx_v)

    def fire_gather(g, b):
        pltpu.async_copy(
            emb_hbm.at[idx_v.at[pl.ds(g * seq, seq)]], rows_v.at[b], sem_g[b])

    def drain_gather(b):
        pltpu.make_async_copy(
            emb_hbm.at[idx_v.at[pl.ds(0, seq)]], rows_v.at[b], sem_g[b]).wait()

    def fire_write(g, b):
        pltpu.async_copy(rows_v.at[b], out_hbm.at[b0 + g], sem_o[b])

    def wait_write(b):
        pltpu.make_async_copy(rows_v.at[b], out_hbm.at[0], sem_o[b]).wait()

    fire_gather(0, 0)
    fire_gather(1, 1)
    drain_gather(0)
    fire_write(0, 0)
    fire_gather(2, 2)
    drain_gather(1)
    fire_write(1, 1)

    n_groups = n_batches
    n_steady = ((n_groups - 3) // 3) * 3

    def body(i, carry):
        for r in range(3):
            g = 3 + 3 * i + r
            b = r
            wait_write(b)
            fire_gather(g, b)
            drain_gather((r + 2) % 3)
            fire_write(g - 1, (r + 2) % 3)
        return carry

    lax.fori_loop(0, n_steady // 3, body, 0)

    for g in range(3 + n_steady, n_groups):
        b = g % 3
        wait_write(b)
        fire_gather(g, b)
        drain_gather((b + 2) % 3)
        fire_write(g - 1, (b + 2) % 3)
    b_last = (n_groups - 1) % 3
    drain_gather(b_last)
    fire_write(n_groups - 1, b_last)
    for db in range(3):
        wait_write((b_last + 1 + db) % 3)


_NCHUNK = 4


def _sc_gather(x_flat, emb_pad, cb, seq, chunk0):
    nw = 32
    n_batches = cb // nw
    mesh = plsc.VectorSubcoreMesh(core_axis_name="c", subcore_axis_name="s")
    kfn = pl.kernel(
        functools.partial(_gather_body, seq=seq, n_batches=n_batches,
                          chunk0=chunk0),
        mesh=mesh,
        out_type=jax.ShapeDtypeStruct((cb, seq, _PAD), jnp.float32),
        scratch_types=[
            pltpu.VMEM((n_batches * seq,), jnp.int32),
            pltpu.VMEM((_NBUF, seq, _PAD), jnp.float32),
        ] + [pltpu.SemaphoreType.DMA] * 6,
        compiler_params=pltpu.CompilerParams(use_tc_tiling_on_sc=True),
    )
    return kfn(x_flat, emb_pad)


@functools.partial(jax.jit, static_argnames=("bsz", "seq"))
def _gather_all(x_flat, emb_pad, bsz, seq):
    cb = bsz // _NCHUNK
    pieces = [_sc_gather(x_flat, emb_pad, cb, seq, c * cb)
              for c in range(_NCHUNK)]
    return jnp.concatenate([p[:, :, :_DIM] for p in pieces], axis=0)


def kernel(x, embedding):
    bsz, seq = x.shape
    x_flat = x.reshape(bsz * seq).astype(jnp.int32)
    emb_pad = jnp.pad(embedding, ((0, 0), (0, _PAD - _DIM)))
    return _gather_all(x_flat, emb_pad, bsz, seq)


# speedup vs baseline: 1.6321x; 1.6321x over previous
import functools

import jax
import jax.numpy as jnp
from jax import lax
from jax.experimental import pallas as pl
from jax.experimental.pallas import tpu as pltpu
from jax.experimental.pallas import tpu_sc as plsc

_DIM = 64
_PAD = 128
_NBUF = 4


def _gather_body(x_hbm, emb_hbm, out_hbm, idx_v, rows_v,
                 sg0, sg1, sg2, sg3, so0, so1, so2, so3, *, seq, n_batches):
    wid = lax.axis_index("s") * 2 + lax.axis_index("c")
    b0 = wid * n_batches
    sem_g = (sg0, sg1, sg2, sg3)
    sem_o = (so0, so1, so2, so3)

    pltpu.sync_copy(x_hbm.at[pl.ds(b0 * seq, n_batches * seq)], idx_v)

    def fire_gather(g, b):
        pltpu.async_copy(
            emb_hbm.at[idx_v.at[pl.ds(g * seq, seq)]], rows_v.at[b], sem_g[b])

    def drain_gather(b):
        pltpu.make_async_copy(
            emb_hbm.at[idx_v.at[pl.ds(0, seq)]], rows_v.at[b], sem_g[b]).wait()

    def fire_write(g, b):
        pltpu.async_copy(rows_v.at[b], out_hbm.at[b0 + g], sem_o[b])

    def wait_write(b):
        pltpu.make_async_copy(rows_v.at[b], out_hbm.at[0], sem_o[b]).wait()

    n_groups = n_batches

    fire_gather(0, 0)
    fire_gather(1, 1)
    fire_gather(2, 2)
    drain_gather(0)
    fire_write(0, 0)
    fire_gather(3, 3)
    drain_gather(1)
    fire_write(1, 1)

    n_steady = ((n_groups - 4) // 4) * 4

    def body(i, carry):
        for r in range(4):
            g = 4 + 4 * i + r
            b = r
            wait_write(b)
            fire_gather(g, b)
            drain_gather((r + 2) % 4)
            fire_write(g - 2, (r + 2) % 4)
        return carry

    lax.fori_loop(0, n_steady // 4, body, 0)

    for g in range(4 + n_steady, n_groups):
        b = g % 4
        wait_write(b)
        fire_gather(g, b)
        drain_gather((b + 2) % 4)
        fire_write(g - 2, (b + 2) % 4)
    g_last = n_groups - 1
    for g in (g_last - 1, g_last):
        b = g % 4
        drain_gather(b)
        fire_write(g, b)
    for g in range(n_groups - 4, n_groups):
        wait_write(g % 4)


@functools.partial(jax.jit, static_argnames=("bsz", "seq"))
def _sc_gather(x_flat, emb_pad, bsz, seq):
    nw = 32
    n_batches = bsz // nw
    mesh = plsc.VectorSubcoreMesh(core_axis_name="c", subcore_axis_name="s")
    kfn = pl.kernel(
        functools.partial(_gather_body, seq=seq, n_batches=n_batches),
        mesh=mesh,
        out_type=jax.ShapeDtypeStruct((bsz, seq, _PAD), jnp.float32),
        scratch_types=[
            pltpu.VMEM((n_batches * seq,), jnp.int32),
            pltpu.VMEM((_NBUF, seq, _PAD), jnp.float32),
        ] + [pltpu.SemaphoreType.DMA] * 8,
        compiler_params=pltpu.CompilerParams(use_tc_tiling_on_sc=True),
    )
    return kfn(x_flat, emb_pad)


def kernel(x, embedding):
    bsz, seq = x.shape
    x_flat = x.reshape(bsz * seq).astype(jnp.int32)
    emb_pad = jnp.pad(embedding, ((0, 0), (0, _PAD - _DIM)))
    out_p = _sc_gather(x_flat, emb_pad, bsz, seq)
    return out_p[:, :, :_DIM]
